# Initial kernel scaffold; baseline (speedup 1.0000x reference)
import jax
import jax.numpy as jnp
from jax.experimental import pallas as pl


def kernel(x, y, edge_index, A2, params, eps_q, eps_p):
    raise NotImplementedError("write your pallas kernel here")


# speedup vs baseline: 1.8941x; 1.8941x over previous
import functools

import jax
import jax.numpy as jnp
from jax.experimental import pallas as pl


def _matT_body(a_ref, v_ref, o_ref):
    @pl.when(pl.program_id(0) == 0)
    def _init():
        o_ref[...] = jnp.zeros_like(o_ref)

    o_ref[...] += jax.lax.dot_general(
        a_ref[...], v_ref[...],
        dimension_numbers=(((0,), (0,)), ((), ())),
        preferred_element_type=jnp.float32)


def _matmul_T(A, V):
    n = A.shape[0]
    c = V.shape[1]
    bm = max(d for d in range(8, min(n, 256) + 1, 8) if n % d == 0)
    return pl.pallas_call(
        _matT_body,
        grid=(n // bm,),
        in_specs=[
            pl.BlockSpec((bm, n), lambda m: (m, 0)),
            pl.BlockSpec((bm, c), lambda m: (m, 0)),
        ],
        out_specs=pl.BlockSpec((n, c), lambda m: (0, 0)),
        out_shape=jax.ShapeDtypeStruct((n, c), jnp.float32),
    )(A, V)


def _gcn_sparse(x, W, b, row, col, n, fill):
    h = x @ W
    deg = jnp.zeros((n,), h.dtype).at[col].add(1.0) + fill
    dis = 1.0 / jnp.sqrt(deg)
    norm = dis[row] * dis[col]
    out = jnp.zeros_like(h).at[col].add(norm[:, None] * h[row])
    out = out + (fill * dis * dis)[:, None] * h
    return out + b


def _bn(x, g, b):
    m = x.mean(axis=0)
    v = x.var(axis=0)
    return (x - m) / jnp.sqrt(v + 1e-5) * g + b


def _unets(plist, xlist, row, col, A2, n):
    nb = len(plist)
    hid = plist[0]['W1'].shape[0]
    k = -(-n // 2)
    x1s, perms, scs = [], [], []
    for p, xin in zip(plist, xlist):
        x1 = jax.nn.relu(_gcn_sparse(xin, p['W0'], p['b0'], row, col, n, 2.0))
        score = jnp.tanh((x1 @ p['p']) / jnp.linalg.norm(p['p']))
        _, perm = jax.lax.top_k(score, k)
        x1s.append(x1)
        perms.append(perm)
        scs.append(score)

    ind = jnp.zeros((n, 8), jnp.float32)
    for t, perm in enumerate(perms):
        ind = ind.at[perm, t].set(1.0)
    cf = _matmul_T(A2, ind)

    diss, us = [], []
    Us = []
    for t, (p, x1, perm, score) in enumerate(zip(plist, x1s, perms, scs)):
        deg = cf[perm, t] + 2.0
        dis = 1.0 / jnp.sqrt(deg)
        xp = x1[perm] * score[perm][:, None]
        h = xp @ p['W1']
        u = dis[:, None] * h
        diss.append(dis)
        us.append(u)
        Us.append(jnp.zeros((n, hid), jnp.float32).at[perm].set(u))
    S = _matmul_T(A2, jnp.concatenate(Us, axis=1) if nb > 1 else Us[0])

    outs = []
    for t, (p, x1, perm, dis, u) in enumerate(zip(plist, x1s, perms, diss, us)):
        s = S[:, t * hid:(t + 1) * hid][perm]
        x2 = jax.nn.relu(dis[:, None] * (s + 2.0 * u) + p['b1'])
        up = jnp.zeros_like(x1).at[perm].set(x2)
        outs.append(_gcn_sparse(x1 + up, p['Wu'], p['bu'], row, col, n, 2.0))
    return outs


def kernel(x, y, edge_index, A2, params, eps_q, eps_p):
    row, col = edge_index[0], edge_index[1]
    n = x.shape[0]
    x_enc, y_post, y_pr = _unets(
        [params['enc'], params['post'], params['prior']], [x, y, x],
        row, col, A2, n)

    mu = _gcn_sparse(y_post, params['Wmu'], params['bmu'], row, col, n, 1.0)
    mu = _bn(mu, params['g_mu'], params['be_mu']).reshape(-1)
    logstd = 0.1 * _gcn_sparse(y_post, params['Wsig'], params['bsig'], row, col, n, 1.0)
    logstd = _bn(logstd, params['g_sig'], params['be_sig']).reshape(-1)
    q = mu + eps_q * jnp.exp(logstd)

    mu_pr = _gcn_sparse(y_pr, params['Wmupr'], params['bmupr'], row, col, n, 1.0)
    mu_pr = _bn(mu_pr, params['g_mupr'], params['be_mupr']).reshape(-1)
    logstd_pr = 0.1 * _gcn_sparse(y_pr, params['Wsigpr'], params['bsigpr'], row, col, n, 1.0)
    logstd_pr = _bn(logstd_pr, params['g_sigpr'], params['be_sigpr']).reshape(-1)

    z = jnp.concatenate([x_enc, q.reshape(x_enc.shape)], axis=1)
    y_hat = jax.nn.sigmoid(_unets([params['dec']], [z], row, col, A2, n)[0])
    return y_hat, (mu, logstd, mu_pr, logstd_pr)
